# baseline (speedup 1.0000x reference)
import functools

import jax
import jax.numpy as jnp
from jax import lax
from jax.experimental import pallas as pl
from jax.experimental.pallas import tpu as pltpu
from jax.experimental.pallas import tpu_sc as plsc

NUM_ATTR = 26
VOCAB = 1000
D = 128
BATCH = 1024
OBJ = 20
FIN = NUM_ATTR * D
N = BATCH * OBJ

NC = 2
NS = 16
NW = NC * NS

CHUNK_ROWS = (2048, 4608, 4608, 4608, 4608)

NP = NUM_ATTR // 2
BM = 512


def _gather(idx, tab, rows_c):
    rows_wc = rows_c // NW
    ns_str = idx.shape[2]
    ch = idx.shape[3]
    mesh = plsc.VectorSubcoreMesh(core_axis_name="c", subcore_axis_name="s")

    @functools.partial(
        pl.kernel,
        out_type=jax.ShapeDtypeStruct((NUM_ATTR, rows_c, D), jnp.float32),
        mesh=mesh,
        scratch_types=[
            pltpu.VMEM((NUM_ATTR, ns_str, ch), jnp.int32),
            pltpu.VMEM((rows_wc, D), jnp.float32),
            pltpu.VMEM((rows_wc, D), jnp.float32),
            pltpu.SemaphoreType.DMA,
            pltpu.SemaphoreType.DMA,
        ],
    )
    def sc_kernel(idx_ref, tab_ref, out_ref, idx_v, rows0, rows1, semg, semw):
        wid = lax.axis_index("s") * NC + lax.axis_index("c")
        base = wid * rows_wc

        def drain_writes():
            pltpu.make_async_copy(tab_ref.at[pl.ds(0, rows_wc)], rows0,
                                  semw).wait()
            pltpu.make_async_copy(tab_ref.at[pl.ds(0, rows_wc)], rows1,
                                  semw).wait()

        pltpu.sync_copy(idx_ref.at[wid], idx_v)

        def fire(a, rows):
            return [
                pltpu.async_copy(tab_ref.at[idx_v.at[a, s]],
                                 rows.at[pl.ds(s * ch, ch)], semg)
                for s in range(ns_str)
            ]

        def body(t, carry):
            @pl.when(t > 0)
            def _():
                drain_writes()

            g0 = fire(2 * t, rows0)
            g1 = fire(2 * t + 1, rows1)
            for g in g0:
                g.wait()
            pltpu.async_copy(rows0, out_ref.at[2 * t, pl.ds(base, rows_wc)],
                             semw)
            for g in g1:
                g.wait()
            pltpu.async_copy(rows1, out_ref.at[2 * t + 1, pl.ds(base, rows_wc)],
                             semw)
            return carry

        lax.fori_loop(0, NUM_ATTR // 2, body, 0)
        drain_writes()

    return sc_kernel(idx, tab)


def _cast_bf16(w):
    def body(w_ref, o_ref):
        o_ref[...] = w_ref[...].astype(jnp.bfloat16)

    return pl.pallas_call(
        body,
        grid=(NUM_ATTR,),
        in_specs=[pl.BlockSpec((D, FIN), lambda i: (i, 0))],
        out_specs=pl.BlockSpec((D, FIN), lambda i: (i, 0)),
        out_shape=jax.ShapeDtypeStruct((FIN, FIN), jnp.bfloat16),
    )(w)


def _mm_body(g_ref, w_ref, b_ref, o_ref):
    dn = (((1,), (1,)), ((), ()))
    acc = jnp.broadcast_to(b_ref[...], (BM, FIN)).astype(jnp.float32)
    for p in range(NP):
        a = jnp.concatenate([g_ref[2 * p], g_ref[2 * p + 1]], axis=-1)
        acc = acc + lax.dot_general(
            a.astype(jnp.bfloat16), w_ref[:, 2 * D * p:2 * D * (p + 1)],
            dn, preferred_element_type=jnp.float32)
    o_ref[...] = acc


def _matmul_chunk(row0, rows_c, g, wb, bias, y_prev):
    blk0 = row0 // BM
    cblk = rows_c // BM
    g_spec = pl.BlockSpec((NUM_ATTR, BM, D), lambda i: (0, i, 0))
    w_spec = pl.BlockSpec((FIN, FIN), lambda i: (0, 0))
    b_spec = pl.BlockSpec((1, FIN), lambda i: (0, 0))
    out_spec = pl.BlockSpec((BM, FIN), lambda i, blk0=blk0: (blk0 + i, 0))
    out_shape = jax.ShapeDtypeStruct((N, FIN), jnp.float32)

    if y_prev is None:
        return pl.pallas_call(
            _mm_body,
            grid=(cblk,),
            in_specs=[g_spec, w_spec, b_spec],
            out_specs=out_spec,
            out_shape=out_shape,
        )(g, wb, bias)

    def body(g_ref, w_ref, b_ref, y_ref, o_ref):
        _mm_body(g_ref, w_ref, b_ref, o_ref)

    return pl.pallas_call(
        body,
        grid=(cblk,),
        in_specs=[g_spec, w_spec, b_spec,
                  pl.BlockSpec(memory_space=pltpu.MemorySpace.HBM)],
        out_specs=out_spec,
        out_shape=out_shape,
        input_output_aliases={3: 0},
    )(g, wb, bias, y_prev)


def _stream_shape(rows_c):
    rows_wc = rows_c // NW
    ns_str = -(-rows_wc // 128)
    ch = rows_wc // ns_str
    assert ns_str * ch == rows_wc and ch % 8 == 0 and ch <= 128
    return ns_str, ch


def kernel(x, tables, W, b):
    xt = x.transpose(2, 1, 0).reshape(NUM_ATTR, N)
    idx = xt + jnp.arange(NUM_ATTR, dtype=jnp.int32)[:, None] * VOCAB
    tab = tables.reshape(NUM_ATTR * VOCAB, D)
    wb = _cast_bf16(W)
    bias = b.reshape(1, FIN)

    gs = []
    row0 = 0
    for rows_c in CHUNK_ROWS:
        ns_str, ch = _stream_shape(rows_c)
        idx_c = (idx[:, row0:row0 + rows_c]
                 .reshape(NUM_ATTR, NW, ns_str, ch).transpose(1, 0, 2, 3))
        gs.append(_gather(idx_c, tab, rows_c))
        row0 += rows_c

    y = None
    row0 = 0
    for rows_c, g in zip(CHUNK_ROWS, gs):
        y = _matmul_chunk(row0, rows_c, g, wb, bias, y)
        row0 += rows_c

    return y.reshape(OBJ, BATCH, FIN).transpose(1, 0, 2)


# speedup vs baseline: 1.0383x; 1.0383x over previous
import functools

import jax
import jax.numpy as jnp
from jax import lax
from jax.experimental import pallas as pl
from jax.experimental.pallas import tpu as pltpu
from jax.experimental.pallas import tpu_sc as plsc

NUM_ATTR = 26
VOCAB = 1000
D = 128
BATCH = 1024
OBJ = 20
FIN = NUM_ATTR * D
N = BATCH * OBJ

NC = 2
NS = 16
NW = NC * NS

CHUNK_ROWS = (4096, 4096, 4096, 4096, 4096)

NP = NUM_ATTR // 2
BM = 512


def _gather(idx, tab, rows_c):
    rows_wc = rows_c // NW
    ns_str = idx.shape[2]
    ch = idx.shape[3]
    mesh = plsc.VectorSubcoreMesh(core_axis_name="c", subcore_axis_name="s")

    @functools.partial(
        pl.kernel,
        out_type=jax.ShapeDtypeStruct((NUM_ATTR, rows_c, D), jnp.float32),
        mesh=mesh,
        scratch_types=[
            pltpu.VMEM((NUM_ATTR, ns_str, ch), jnp.int32),
            pltpu.VMEM((rows_wc, D), jnp.float32),
            pltpu.VMEM((rows_wc, D), jnp.float32),
            pltpu.SemaphoreType.DMA,
            pltpu.SemaphoreType.DMA,
        ],
    )
    def sc_kernel(idx_ref, tab_ref, out_ref, idx_v, rows0, rows1, semg, semw):
        wid = lax.axis_index("s") * NC + lax.axis_index("c")
        base = wid * rows_wc

        def drain_writes():
            pltpu.make_async_copy(tab_ref.at[pl.ds(0, rows_wc)], rows0,
                                  semw).wait()
            pltpu.make_async_copy(tab_ref.at[pl.ds(0, rows_wc)], rows1,
                                  semw).wait()

        pltpu.sync_copy(idx_ref.at[wid], idx_v)

        def fire(a, rows):
            return [
                pltpu.async_copy(tab_ref.at[idx_v.at[a, s]],
                                 rows.at[pl.ds(s * ch, ch)], semg)
                for s in range(ns_str)
            ]

        def body(t, carry):
            @pl.when(t > 0)
            def _():
                drain_writes()

            g0 = fire(2 * t, rows0)
            g1 = fire(2 * t + 1, rows1)
            for g in g0:
                g.wait()
            pltpu.async_copy(rows0, out_ref.at[2 * t, pl.ds(base, rows_wc)],
                             semw)
            for g in g1:
                g.wait()
            pltpu.async_copy(rows1, out_ref.at[2 * t + 1, pl.ds(base, rows_wc)],
                             semw)
            return carry

        lax.fori_loop(0, NUM_ATTR // 2, body, 0)
        drain_writes()

    return sc_kernel(idx, tab)


def _cast_bf16(w):
    def body(w_ref, o_ref):
        o_ref[...] = w_ref[...].astype(jnp.bfloat16)

    return pl.pallas_call(
        body,
        grid=(NUM_ATTR,),
        in_specs=[pl.BlockSpec((D, FIN), lambda i: (i, 0))],
        out_specs=pl.BlockSpec((D, FIN), lambda i: (i, 0)),
        out_shape=jax.ShapeDtypeStruct((FIN, FIN), jnp.bfloat16),
    )(w)


def _mm_body(g_ref, w_ref, b_ref, o_ref):
    dn = (((1,), (1,)), ((), ()))
    acc = jnp.broadcast_to(b_ref[...], (BM, FIN)).astype(jnp.float32)
    for p in range(NP):
        a = jnp.concatenate([g_ref[2 * p], g_ref[2 * p + 1]], axis=-1)
        acc = acc + lax.dot_general(
            a.astype(jnp.bfloat16), w_ref[:, 2 * D * p:2 * D * (p + 1)],
            dn, preferred_element_type=jnp.float32)
    o_ref[...] = acc


def _matmul_chunk(row0, rows_c, g, wb, bias, y_prev):
    blk0 = row0 // BM
    cblk = rows_c // BM
    g_spec = pl.BlockSpec((NUM_ATTR, BM, D), lambda i: (0, i, 0))
    w_spec = pl.BlockSpec((FIN, FIN), lambda i: (0, 0))
    b_spec = pl.BlockSpec((1, FIN), lambda i: (0, 0))
    out_spec = pl.BlockSpec((BM, FIN), lambda i, blk0=blk0: (blk0 + i, 0))
    out_shape = jax.ShapeDtypeStruct((N, FIN), jnp.float32)

    if y_prev is None:
        return pl.pallas_call(
            _mm_body,
            grid=(cblk,),
            in_specs=[g_spec, w_spec, b_spec],
            out_specs=out_spec,
            out_shape=out_shape,
        )(g, wb, bias)

    def body(g_ref, w_ref, b_ref, y_ref, o_ref):
        _mm_body(g_ref, w_ref, b_ref, o_ref)

    return pl.pallas_call(
        body,
        grid=(cblk,),
        in_specs=[g_spec, w_spec, b_spec,
                  pl.BlockSpec(memory_space=pltpu.MemorySpace.HBM)],
        out_specs=out_spec,
        out_shape=out_shape,
        input_output_aliases={3: 0},
    )(g, wb, bias, y_prev)


def _stream_shape(rows_c):
    rows_wc = rows_c // NW
    ns_str = -(-rows_wc // 128)
    ch = rows_wc // ns_str
    assert ns_str * ch == rows_wc and ch % 8 == 0 and ch <= 128
    return ns_str, ch


def kernel(x, tables, W, b):
    xt = x.transpose(2, 1, 0).reshape(NUM_ATTR, N)
    idx = xt + jnp.arange(NUM_ATTR, dtype=jnp.int32)[:, None] * VOCAB
    tab = tables.reshape(NUM_ATTR * VOCAB, D)
    wb = _cast_bf16(W)
    bias = b.reshape(1, FIN)

    gs = []
    row0 = 0
    for rows_c in CHUNK_ROWS:
        ns_str, ch = _stream_shape(rows_c)
        idx_c = (idx[:, row0:row0 + rows_c]
                 .reshape(NUM_ATTR, NW, ns_str, ch).transpose(1, 0, 2, 3))
        gs.append(_gather(idx_c, tab, rows_c))
        row0 += rows_c

    y = None
    row0 = 0
    for rows_c, g in zip(CHUNK_ROWS, gs):
        y = _matmul_chunk(row0, rows_c, g, wb, bias, y)
        row0 += rows_c

    return y.reshape(OBJ, BATCH, FIN).transpose(1, 0, 2)
